# Initial kernel scaffold; baseline (speedup 1.0000x reference)
import jax
import jax.numpy as jnp
from jax.experimental import pallas as pl


def kernel(x, edge_index, W1, b1, lg, lb, W2, b2, t, ng, nb):
    raise NotImplementedError("write your pallas kernel here")


# speedup vs baseline: 7.0337x; 7.0337x over previous
import functools

import jax
import jax.numpy as jnp
from jax import lax
from jax.experimental import pallas as pl
from jax.experimental.pallas import tpu as pltpu
from jax.experimental.pallas import tpu_sc as plsc

N = 10000
D = 128
H = 256
NLAYERS = 4
E = 320000
EPS_MSG = 1e-7
LN_EPS = 1e-5

NCORES = 2
NTILES = 16
CHUNK = 128
SB = 40
NSB = 4
NCHUNK = SB * NSB
EPT = CHUNK * NCHUNK
E_PAD = EPT * NTILES
ACC_ROWS = 10112
ZSLICE = ACC_ROWS // NTILES
WSLICE = 624
WTAIL = N - NTILES * WSLICE

BLK = 2000


def _edge_accumulate(tbl_hbm, out_hbm, tid, sidx_hbm, didx_hbm, sidx_v,
                     didx_v, buf0, buf1, acc_sh, sem0, sem1):
    def _zb(k, carry):
        buf0[k // 8, pl.ds((k % 8) * 16, 16)] = jnp.zeros((16,), jnp.float32)
        return carry

    lax.fori_loop(0, CHUNK * 8, _zb, 0)
    zbase = tid * ZSLICE
    for k in range(ZSLICE // CHUNK):
        pltpu.sync_copy(buf0, acc_sh.at[pl.ds(zbase + k * CHUNK, CHUNK)])
    rem = ZSLICE % CHUNK
    pltpu.sync_copy(buf0.at[pl.ds(0, rem)],
                    acc_sh.at[pl.ds(zbase + (ZSLICE // CHUNK) * CHUNK, rem)])
    plsc.subcore_barrier()

    def _superblock(sb, carry):
        pltpu.sync_copy(sidx_hbm.at[tid, pl.ds(sb * SB, SB)], sidx_v)
        pltpu.sync_copy(didx_hbm.at[tid, pl.ds(sb * SB, SB)], didx_v)
        pltpu.async_copy(tbl_hbm.at[sidx_v.at[0]], buf0, sem0)
        pltpu.async_copy(tbl_hbm.at[sidx_v.at[1]], buf1, sem1)

        def _body(j, carry2):
            c0 = 2 * j
            pltpu.make_async_copy(tbl_hbm.at[sidx_v.at[c0]], buf0, sem0).wait()
            pltpu.sync_copy(buf0, acc_sh.at[didx_v.at[c0]], add=True)

            @pl.when(j < SB // 2 - 1)
            def _():
                pltpu.async_copy(tbl_hbm.at[sidx_v.at[c0 + 2]], buf0, sem0)

            pltpu.make_async_copy(tbl_hbm.at[sidx_v.at[c0 + 1]], buf1,
                                  sem1).wait()
            pltpu.sync_copy(buf1, acc_sh.at[didx_v.at[c0 + 1]], add=True)

            @pl.when(j < SB // 2 - 1)
            def _():
                pltpu.async_copy(tbl_hbm.at[sidx_v.at[c0 + 3]], buf1, sem1)

            return carry2

        lax.fori_loop(0, SB // 2, _body, 0)
        return carry

    lax.fori_loop(0, NSB, _superblock, 0)
    plsc.subcore_barrier()
    wbase = tid * WSLICE
    pltpu.sync_copy(acc_sh.at[pl.ds(wbase, WSLICE)],
                    out_hbm.at[pl.ds(wbase, WSLICE)])

    @pl.when(tid == 0)
    def _():
        pltpu.sync_copy(acc_sh.at[pl.ds(NTILES * WSLICE, WTAIL)],
                        out_hbm.at[pl.ds(NTILES * WSLICE, WTAIL)])


def _sc_edge_body(p_hbm, q_hbm, sidx_hbm, didx_hbm, den_hbm, s_hbm,
                  sidx_v, didx_v, buf0, buf1, acc_sh, sem0, sem1):
    c = lax.axis_index("c")
    tid = lax.axis_index("s")

    @pl.when(c == 0)
    def _():
        _edge_accumulate(p_hbm, den_hbm, tid, sidx_hbm, didx_hbm, sidx_v,
                         didx_v, buf0, buf1, acc_sh, sem0, sem1)

    @pl.when(c == 1)
    def _():
        _edge_accumulate(q_hbm, s_hbm, tid, sidx_hbm, didx_hbm, sidx_v,
                         didx_v, buf0, buf1, acc_sh, sem0, sem1)


@functools.cache
def _build_sc_edge():
    return pl.kernel(
        _sc_edge_body,
        out_type=(jax.ShapeDtypeStruct((N, D), jnp.float32),
                  jax.ShapeDtypeStruct((N, D), jnp.float32)),
        mesh=plsc.VectorSubcoreMesh(core_axis_name="c", subcore_axis_name="s",
                                    num_cores=NCORES, num_subcores=NTILES),
        scratch_types=[
            pltpu.VMEM((SB, CHUNK), jnp.int32),
            pltpu.VMEM((SB, CHUNK), jnp.int32),
            pltpu.VMEM((CHUNK, D), jnp.float32),
            pltpu.VMEM((CHUNK, D), jnp.float32),
            pltpu.VMEM_SHARED((ACC_ROWS, D), jnp.float32),
            pltpu.SemaphoreType.DMA,
            pltpu.SemaphoreType.DMA,
        ],
    )


def _sc_edge(p, q, sidx, didx):
    return _build_sc_edge()(p, q, sidx, didx)


def _ln(h, g, b):
    m = jnp.mean(h, axis=-1, keepdims=True)
    v = jnp.mean((h - m) ** 2, axis=-1, keepdims=True)
    return (h - m) * lax.rsqrt(v + LN_EPS) * g + b


def _tc_first_body(x_ref, tn_ref, p_ref, q_ref):
    r = jnp.maximum(x_ref[...], 0.0) + EPS_MSG
    e = jnp.exp(r * tn_ref[...])
    p_ref[...] = e
    q_ref[...] = r * e


def _tc_mid_body(first, den_ref, s_ref, z_ref, *refs):
    if first:
        (w1_ref, b1_ref, lg_ref, lb_ref, w2_ref, b2_ref, ngn_ref, nbn_ref,
         tn_ref, h_ref, zn_ref, p_ref, q_ref) = refs
        hprev = 0.0
    else:
        (hprev_ref, w1_ref, b1_ref, lg_ref, lb_ref, w2_ref, b2_ref, ngn_ref,
         nbn_ref, tn_ref, h_ref, zn_ref, p_ref, q_ref) = refs
        hprev = hprev_ref[...]
    aggr = s_ref[...] / (den_ref[...] + 1e-16)
    u = aggr + z_ref[...]
    h1 = jnp.dot(u, w1_ref[...], preferred_element_type=jnp.float32) + b1_ref[...]
    h1 = jnp.maximum(_ln(h1, lg_ref[...], lb_ref[...]), 0.0)
    h2 = jnp.dot(h1, w2_ref[...], preferred_element_type=jnp.float32) + b2_ref[...]
    h = hprev + h2
    h_ref[...] = h
    zn = jnp.maximum(_ln(h, ngn_ref[...], nbn_ref[...]), 0.0)
    zn_ref[...] = zn
    r = zn + EPS_MSG
    e = jnp.exp(r * tn_ref[...])
    p_ref[...] = e
    q_ref[...] = r * e


def _tc_last_body(den_ref, s_ref, z_ref, hprev_ref, w1_ref, b1_ref, lg_ref,
                  lb_ref, w2_ref, b2_ref, ng0_ref, nb0_ref, out_ref):
    aggr = s_ref[...] / (den_ref[...] + 1e-16)
    u = aggr + z_ref[...]
    h1 = jnp.dot(u, w1_ref[...], preferred_element_type=jnp.float32) + b1_ref[...]
    h1 = jnp.maximum(_ln(h1, lg_ref[...], lb_ref[...]), 0.0)
    h2 = jnp.dot(h1, w2_ref[...], preferred_element_type=jnp.float32) + b2_ref[...]
    h = hprev_ref[...] + h2
    out_ref[...] = jnp.maximum(_ln(h, ng0_ref[...], nb0_ref[...]), 0.0)


def _row_spec(cols):
    return pl.BlockSpec((BLK, cols), lambda i: (i, 0))


def _bcast_spec(rows, cols):
    return pl.BlockSpec((rows, cols), lambda i: (0, 0))


_N_SPEC = _row_spec(D)
_W1_SPEC = _bcast_spec(D, H)
_W2_SPEC = _bcast_spec(H, D)
_VH_SPEC = _bcast_spec(1, H)
_VD_SPEC = _bcast_spec(1, D)

_out_nd = jax.ShapeDtypeStruct((N, D), jnp.float32)

_tc_first = pl.pallas_call(
    _tc_first_body,
    grid=(N // BLK,),
    in_specs=[_N_SPEC, _VD_SPEC],
    out_specs=(_N_SPEC, _N_SPEC),
    out_shape=(_out_nd, _out_nd),
)

_MID_PARAM_SPECS = [_W1_SPEC, _VH_SPEC, _VH_SPEC, _VH_SPEC, _W2_SPEC,
                    _VD_SPEC, _VD_SPEC, _VD_SPEC, _VD_SPEC]

_tc_mid_first = pl.pallas_call(
    functools.partial(_tc_mid_body, True),
    grid=(N // BLK,),
    in_specs=[_N_SPEC, _N_SPEC, _N_SPEC] + _MID_PARAM_SPECS,
    out_specs=(_N_SPEC, _N_SPEC, _N_SPEC, _N_SPEC),
    out_shape=(_out_nd, _out_nd, _out_nd, _out_nd),
)

_tc_mid = pl.pallas_call(
    functools.partial(_tc_mid_body, False),
    grid=(N // BLK,),
    in_specs=[_N_SPEC, _N_SPEC, _N_SPEC, _N_SPEC] + _MID_PARAM_SPECS,
    out_specs=(_N_SPEC, _N_SPEC, _N_SPEC, _N_SPEC),
    out_shape=(_out_nd, _out_nd, _out_nd, _out_nd),
)

_tc_last = pl.pallas_call(
    _tc_last_body,
    grid=(N // BLK,),
    in_specs=[_N_SPEC, _N_SPEC, _N_SPEC, _N_SPEC, _W1_SPEC, _VH_SPEC,
              _VH_SPEC, _VH_SPEC, _W2_SPEC, _VD_SPEC, _VD_SPEC, _VD_SPEC],
    out_specs=_N_SPEC,
    out_shape=_out_nd,
)


def kernel(x, edge_index, W1, b1, lg, lb, W2, b2, t, ng, nb):
    src = edge_index[0]
    dst = edge_index[1]
    pad = E_PAD - E
    sidx = jnp.concatenate([src, jnp.zeros((pad,), jnp.int32)])
    didx = jnp.concatenate([dst, jnp.full((pad,), N, jnp.int32)])
    sidx = sidx.reshape(NTILES, NCHUNK, CHUNK)
    didx = didx.reshape(NTILES, NCHUNK, CHUNK)

    b1r = b1.reshape(NLAYERS, 1, H)
    lgr = lg.reshape(NLAYERS, 1, H)
    lbr = lb.reshape(NLAYERS, 1, H)
    b2r = b2.reshape(NLAYERS, 1, D)
    ngr = ng.reshape(NLAYERS, 1, D)
    nbr = nb.reshape(NLAYERS, 1, D)
    tr = jnp.broadcast_to(t.reshape(NLAYERS, 1, 1), (NLAYERS, 1, D))

    p, q = _tc_first(x, tr[0])
    z = x
    h = None
    for i in range(NLAYERS):
        den, s = _sc_edge(p, q, sidx, didx)
        params = (W1[i], b1r[i], lgr[i], lbr[i], W2[i], b2r[i])
        if i == 0:
            h, z, p, q = _tc_mid_first(den, s, z, *params,
                                       ngr[1], nbr[1], tr[1])
        elif i < NLAYERS - 1:
            h, z, p, q = _tc_mid(den, s, z, h, *params,
                                 ngr[i + 1], nbr[i + 1], tr[i + 1])
        else:
            out = _tc_last(den, s, z, h, *params, ngr[0], nbr[0])
    return out
